# baseline (speedup 1.0000x reference)
import functools
import math

import jax
import jax.numpy as jnp
from jax import lax
from jax.experimental import pallas as pl
from jax.experimental.pallas import tpu as pltpu
from jax.experimental.pallas import tpu_sc as plsc

S = 64.0
M = 0.5
COS_M = math.cos(M)
SIN_M = math.sin(M)

_B = 1024
_C = 100000
_BC = 2048

_NW = 32
_RPW = _B // _NW
_L = 16


def _sc_body(cos_flat, label_hbm, cstar_hbm, lab_v, idx_v, val_v, sem):
    wid = lax.axis_index("s") * 2 + lax.axis_index("c")
    base = wid * _RPW
    pltpu.sync_copy(label_hbm.at[pl.ds(base, _RPW)], lab_v)
    for t in range(_RPW // _L):
        lab = lab_v[pl.ds(t * _L, _L)]
        rows = lax.iota(jnp.int32, _L) + (base + t * _L)
        idx_v[pl.ds(t * _L, _L)] = rows * _C + jnp.maximum(lab, 0)
    pltpu.async_copy(cos_flat.at[idx_v], val_v, sem).wait()
    pltpu.sync_copy(val_v, cstar_hbm.at[pl.ds(base, _RPW)])


_sc_gather = functools.partial(
    pl.kernel,
    out_type=jax.ShapeDtypeStruct((_B,), jnp.float32),
    mesh=plsc.VectorSubcoreMesh(core_axis_name="c", subcore_axis_name="s"),
    scratch_types=[
        pltpu.VMEM((_RPW,), jnp.int32),
        pltpu.VMEM((_RPW,), jnp.int32),
        pltpu.VMEM((_RPW,), jnp.float32),
        pltpu.SemaphoreType.DMA,
    ],
)(_sc_body)


def _tc_block(cos_ref, lab_ref, cstar_ref, out_ref):
    j = pl.program_id(0)
    c = cos_ref[...]
    cs = cstar_ref[...]
    corr = (cs * COS_M - jnp.sqrt(jnp.maximum(1.0 - cs * cs, 0.0)) * SIN_M) * S
    cols = lax.broadcasted_iota(jnp.int32, c.shape, 1) + j * _BC
    mask = cols == lab_ref[...]
    out_ref[...] = jnp.where(mask, corr, c * S)


@jax.jit
def kernel(cosine, label):
    B, C = cosine.shape
    lab32 = label.astype(jnp.int32)
    cstar = _sc_gather(cosine.reshape(-1), lab32)
    grid = (pl.cdiv(C, _BC),)
    return pl.pallas_call(
        _tc_block,
        grid=grid,
        in_specs=[
            pl.BlockSpec((B, _BC), lambda j: (0, j)),
            pl.BlockSpec((B, 1), lambda j: (0, 0)),
            pl.BlockSpec((B, 1), lambda j: (0, 0)),
        ],
        out_specs=pl.BlockSpec((B, _BC), lambda j: (0, j)),
        out_shape=jax.ShapeDtypeStruct((B, C), cosine.dtype),
    )(cosine, lab32.reshape(B, 1), cstar.reshape(B, 1))


# speedup vs baseline: 1.5862x; 1.5862x over previous
import functools
import math

import jax
import jax.numpy as jnp
from jax import lax
from jax.experimental import pallas as pl
from jax.experimental.pallas import tpu as pltpu
from jax.experimental.pallas import tpu_sc as plsc

S = 64.0
M = 0.5
COS_M = math.cos(M)
SIN_M = math.sin(M)

_B = 1024
_C = 100000
_BC = 2048

_NW = 32
_RPW = _B // _NW
_L = 16


def _sc_body(cos_hbm, label_hbm, cstar_hbm, lab_v, val16_v, sem):
    wid = lax.axis_index("s") * 2 + lax.axis_index("c")
    base = wid * _RPW
    pltpu.sync_copy(label_hbm.at[pl.ds(base, _RPW)], lab_v)
    descs = []
    for t in range(_RPW // _L):
        labs = jnp.maximum(lab_v[pl.ds(t * _L, _L)], 0)
        for lane in range(_L):
            r = t * _L + lane
            col0 = (labs[lane] // _L) * _L
            descs.append(
                pltpu.async_copy(
                    cos_hbm.at[base + r, pl.ds(col0, _L)],
                    val16_v.at[r],
                    sem,
                )
            )
    for d in descs:
        d.wait()
    pltpu.sync_copy(val16_v, cstar_hbm.at[pl.ds(base, _RPW)])


_sc_gather = functools.partial(
    pl.kernel,
    out_type=jax.ShapeDtypeStruct((_B, _L), jnp.float32),
    mesh=plsc.VectorSubcoreMesh(core_axis_name="c", subcore_axis_name="s"),
    scratch_types=[
        pltpu.VMEM((_RPW,), jnp.int32),
        pltpu.VMEM((_RPW, _L), jnp.float32),
        pltpu.SemaphoreType.DMA,
    ],
)(_sc_body)


def _tc_block(cos_ref, lab_ref, chunk_ref, out_ref):
    j = pl.program_id(0)
    c = cos_ref[...]
    lab = lab_ref[...]
    chunk = chunk_ref[...]
    lanes = lax.broadcasted_iota(jnp.int32, chunk.shape, 1)
    sel = lanes == (lab - (lab // _L) * _L)
    cs = jnp.sum(jnp.where(sel, chunk, 0.0), axis=1, keepdims=True)
    corr = (cs * COS_M - jnp.sqrt(jnp.maximum(1.0 - cs * cs, 0.0)) * SIN_M) * S
    cols = lax.broadcasted_iota(jnp.int32, c.shape, 1) + j * _BC
    mask = cols == lab
    out_ref[...] = jnp.where(mask, corr, c * S)


@jax.jit
def kernel(cosine, label):
    B, C = cosine.shape
    lab32 = label.astype(jnp.int32)
    cstar = _sc_gather(cosine, lab32)
    grid = (pl.cdiv(C, _BC),)
    return pl.pallas_call(
        _tc_block,
        grid=grid,
        in_specs=[
            pl.BlockSpec((B, _BC), lambda j: (0, j)),
            pl.BlockSpec((B, 1), lambda j: (0, 0)),
            pl.BlockSpec((B, _L), lambda j: (0, 0)),
        ],
        out_specs=pl.BlockSpec((B, _BC), lambda j: (0, j)),
        out_shape=jax.ShapeDtypeStruct((B, C), cosine.dtype),
    )(cosine, lab32.reshape(B, 1), cstar)
